# Initial kernel scaffold; baseline (speedup 1.0000x reference)
import jax
import jax.numpy as jnp
from jax.experimental import pallas as pl


def kernel(x, emb):
    raise NotImplementedError("write your pallas kernel here")


# speedup vs baseline: 1.6758x; 1.6758x over previous
import jax
import jax.numpy as jnp
from jax.experimental import pallas as pl
from jax.experimental.pallas import tpu as pltpu

DIM_ = 1024
BLK_L = 512


def _body(x_ref, emb_ref, o_ref):
    o_ref[...] = x_ref[...] + emb_ref[...][None, :, :]


def kernel(x, emb):
    B, L, D = x.shape
    grid = (L // BLK_L, B)
    return pl.pallas_call(
        _body,
        grid=grid,
        in_specs=[
            pl.BlockSpec((1, BLK_L, D), lambda i, j: (j, i, 0)),
            pl.BlockSpec((BLK_L, D), lambda i, j: (i, 0)),
        ],
        out_specs=pl.BlockSpec((1, BLK_L, D), lambda i, j: (j, i, 0)),
        out_shape=jax.ShapeDtypeStruct((B, L, D), x.dtype),
    )(x, emb)


# speedup vs baseline: 1.8761x; 1.1195x over previous
import jax
import jax.numpy as jnp
from jax.experimental import pallas as pl
from jax.experimental.pallas import tpu as pltpu

DIM_ = 1024
BLK_L = 1024


def _body(x_ref, emb_ref, o_ref):
    o_ref[...] = x_ref[...] + emb_ref[...][None, :, :]


def kernel(x, emb):
    B, L, D = x.shape
    grid = (L // BLK_L, B)
    return pl.pallas_call(
        _body,
        grid=grid,
        in_specs=[
            pl.BlockSpec((1, BLK_L, D), lambda i, j: (j, i, 0)),
            pl.BlockSpec((BLK_L, D), lambda i, j: (i, 0)),
        ],
        out_specs=pl.BlockSpec((1, BLK_L, D), lambda i, j: (j, i, 0)),
        out_shape=jax.ShapeDtypeStruct((B, L, D), x.dtype),
    )(x, emb)


# speedup vs baseline: 1.9985x; 1.0653x over previous
import jax
import jax.numpy as jnp
from jax.experimental import pallas as pl
from jax.experimental.pallas import tpu as pltpu

DIM_ = 1024
BLK_L = 2048


def _body(x_ref, emb_ref, o_ref):
    o_ref[...] = x_ref[...] + emb_ref[...][None, :, :]


def kernel(x, emb):
    B, L, D = x.shape
    grid = (L // BLK_L, B)
    return pl.pallas_call(
        _body,
        grid=grid,
        in_specs=[
            pl.BlockSpec((1, BLK_L, D), lambda i, j: (j, i, 0)),
            pl.BlockSpec((BLK_L, D), lambda i, j: (i, 0)),
        ],
        out_specs=pl.BlockSpec((1, BLK_L, D), lambda i, j: (j, i, 0)),
        out_shape=jax.ShapeDtypeStruct((B, L, D), x.dtype),
    )(x, emb)
